# baseline (speedup 1.0000x reference)
import functools

import jax
import jax.numpy as jnp
from jax import lax
from jax.experimental import pallas as pl
from jax.experimental.pallas import tpu as pltpu
from jax.experimental.pallas import tpu_sc as plsc

NUM_FACTORS = 32
BATCH = 16384
NC = 2
NS = 16
L = 16
NW = NC * NS
B_PER_W = BATCH // NW
IDX_CHUNK = 128
N_CHUNKS = B_PER_W // IDX_CHUNK
N_GROUPS = B_PER_W // L


def _mf_body(uid_hbm, aid_hbm, uemb_hbm, aemb_hbm, ubias_hbm, abias_hbm,
             out_hbm, uidx_v, aidx_v, urows_v, arows_v, ub_v, ab_v, out_v,
             sem):
    wid = lax.axis_index("s") * NC + lax.axis_index("c")
    base = wid * B_PER_W

    pltpu.sync_copy(uid_hbm.at[wid], uidx_v)
    pltpu.sync_copy(aid_hbm.at[wid], aidx_v)

    copies = []
    for j in range(N_CHUNKS):
        dst = pl.ds(j * IDX_CHUNK, IDX_CHUNK)
        copies.append(pltpu.async_copy(uemb_hbm.at[uidx_v.at[j]],
                                       urows_v.at[dst], sem))
        copies.append(pltpu.async_copy(aemb_hbm.at[aidx_v.at[j]],
                                       arows_v.at[dst], sem))
        copies.append(pltpu.async_copy(ubias_hbm.at[uidx_v.at[j]],
                                       ub_v.at[dst], sem))
        copies.append(pltpu.async_copy(abias_hbm.at[aidx_v.at[j]],
                                       ab_v.at[dst], sem))
    for c in copies:
        c.wait()

    iota16 = lax.iota(jnp.int32, L)

    def group(g, carry):
        rows = g * L + iota16
        acc = ub_v[pl.ds(g * L, L)] + ab_v[pl.ds(g * L, L)]
        for j in range(NUM_FACTORS):
            cols = jnp.full((L,), j, jnp.int32)
            uu = plsc.load_gather(urows_v, [rows, cols])
            aa = plsc.load_gather(arows_v, [rows, cols])
            acc = acc + uu * aa
        out_v[pl.ds(g * L, L)] = acc
        return carry

    lax.fori_loop(0, N_GROUPS, group, 0)

    pltpu.sync_copy(out_v, out_hbm.at[pl.ds(base, B_PER_W)])


@jax.jit
def _mf_kernel(uids, aids, user_embeddings, anime_embeddings,
               user_biases, anime_biases):
    mesh = plsc.VectorSubcoreMesh(core_axis_name="c", subcore_axis_name="s",
                                  num_cores=NC, num_subcores=NS)
    return pl.kernel(
        _mf_body,
        out_type=jax.ShapeDtypeStruct((BATCH,), jnp.float32),
        mesh=mesh,
        compiler_params=pltpu.CompilerParams(needs_layout_passes=False,
                                             use_tc_tiling_on_sc=False),
        scratch_types=[
            pltpu.VMEM((N_CHUNKS, IDX_CHUNK), jnp.int32),
            pltpu.VMEM((N_CHUNKS, IDX_CHUNK), jnp.int32),
            pltpu.VMEM((B_PER_W, NUM_FACTORS), jnp.float32),
            pltpu.VMEM((B_PER_W, NUM_FACTORS), jnp.float32),
            pltpu.VMEM((B_PER_W,), jnp.float32),
            pltpu.VMEM((B_PER_W,), jnp.float32),
            pltpu.VMEM((B_PER_W,), jnp.float32),
            pltpu.SemaphoreType.DMA,
        ],
    )(uids, aids, user_embeddings, anime_embeddings,
      user_biases, anime_biases)


def kernel(userIds, animeIds, user_embeddings, anime_embeddings,
           user_biases, anime_biases):
    uids = userIds.astype(jnp.int32).reshape(NW, N_CHUNKS, IDX_CHUNK)
    aids = animeIds.astype(jnp.int32).reshape(NW, N_CHUNKS, IDX_CHUNK)
    ub = user_biases.reshape(-1)
    ab = anime_biases.reshape(-1)
    return _mf_kernel(uids, aids, user_embeddings, anime_embeddings, ub, ab)
